# baseline (speedup 1.0000x reference)
import jax
import jax.numpy as jnp
from jax import lax
from jax.experimental import pallas as pl
from jax.experimental.pallas import tpu as pltpu
from jax.experimental.pallas import tpu_sc as plsc

_N = 131072
_K = 512
_D = 32
_BETA = 0.25
_BN = 4096
_G = _N // _BN

_NC = 2
_NS = 16
_NW = _NC * _NS
_NH = _N // 4
_BPW = _NH // _NW
_CH = 512
_NCH = _BPW // _CH


def _dist_body(z_ref, emb_ref, inds_ref, loss_ref):
    i = pl.program_id(0)
    z = z_ref[...]
    emb = emb_ref[...]
    e_sq = jnp.sum(emb * emb, axis=1, keepdims=True)
    z_sq = jnp.sum(z * z, axis=1, keepdims=True)
    z_sq_row = jax.lax.transpose(z_sq, (1, 0))
    dist = (z_sq_row + e_sq) - 2.0 * jax.lax.dot_general(
        emb.astype(jnp.bfloat16), z.astype(jnp.bfloat16),
        (((1,), (1,)), ((), ())),
        preferred_element_type=jnp.float32)
    row = jax.lax.broadcasted_iota(jnp.int32, (_K, _BN), 0)
    dmin = jnp.min(dist, axis=0)
    inds = jnp.min(jnp.where(dist == dmin[None, :], row, _K),
                   axis=0).astype(jnp.int32)
    inds_ref[0, 0, :] = inds
    partial = jnp.sum(dmin).reshape(1, 1)

    @pl.when(i == 0)
    def _():
        loss_ref[...] = jnp.zeros((1, 1), jnp.float32)

    loss_ref[...] += partial


_NSPLIT = 4
_G2 = _G // _NSPLIT


def _make_tc_dist(split):
    return pl.pallas_call(
        _dist_body,
        grid=(_G2,),
        in_specs=[
            pl.BlockSpec((_BN, _D), lambda i: (i + split * _G2, 0)),
            pl.BlockSpec((_K, _D), lambda i: (0, 0)),
        ],
        out_specs=[
            pl.BlockSpec((1, 1, _BN), lambda i: (i, 0, 0)),
            pl.BlockSpec((1, 1), lambda i: (0, 0)),
        ],
        out_shape=[
            jax.ShapeDtypeStruct((_G2, 1, _BN), jnp.int32),
            jax.ShapeDtypeStruct((1, 1), jnp.float32),
        ],
    )


_tc_dists = [_make_tc_dist(s) for s in range(_NSPLIT)]


def _gather_body(emb_hbm, idx_hbm, out_hbm, idx_v, rows_v, sem_i, sem_g, sem_s):
    wid = lax.axis_index("s") * _NC + lax.axis_index("c")
    base = wid * _BPW
    pltpu.async_copy(idx_hbm.at[pl.ds(base, _CH)], idx_v.at[0], sem_i).wait()
    store_handles = []
    for c in range(_NCH):
        cur = c % 2
        if c + 1 < _NCH:
            nxt_off = base + (c + 1) * _CH
            h_idx = pltpu.async_copy(
                idx_hbm.at[pl.ds(nxt_off, _CH)], idx_v.at[1 - cur], sem_i)
        if c >= 2:
            store_handles[c - 2].wait()
        pltpu.async_copy(emb_hbm.at[idx_v.at[cur]], rows_v.at[cur],
                         sem_g).wait()
        store_handles.append(pltpu.async_copy(
            rows_v.at[cur], out_hbm.at[pl.ds(base + c * _CH, _CH)], sem_s))
        if c + 1 < _NCH:
            h_idx.wait()
    for h in store_handles[-2:]:
        h.wait()


_sc_gather = pl.kernel(
    _gather_body,
    out_type=jax.ShapeDtypeStruct((_NH, _D), jnp.float32),
    mesh=plsc.VectorSubcoreMesh(core_axis_name="c", subcore_axis_name="s"),
    scratch_types=[
        pltpu.VMEM((2, _CH), jnp.int32),
        pltpu.VMEM((2, _CH, _D), jnp.float32),
        pltpu.SemaphoreType.DMA,
        pltpu.SemaphoreType.DMA,
        pltpu.SemaphoreType.DMA,
    ],
    compiler_params=pltpu.CompilerParams(use_tc_tiling_on_sc=False),
)


def kernel(z_e, embeddings):
    zq_parts, inds_parts = [], []
    loss_acc = None
    for s in range(_NSPLIT):
        inds3, lacc = _tc_dists[s](z_e, embeddings)
        inds_s = inds3.reshape(_NH)
        zq_parts.append(_sc_gather(embeddings, inds_s))
        inds_parts.append(inds_s)
        loss_acc = lacc if loss_acc is None else loss_acc + lacc
    zq = jnp.concatenate(zq_parts, axis=0)
    inds = jnp.concatenate(inds_parts, axis=0)
    loss = loss_acc[0, 0] * ((1.0 + _BETA) / (_N * _D))
    return (zq, inds, loss)


# speedup vs baseline: 1.1224x; 1.1224x over previous
import jax
import jax.numpy as jnp
from jax import lax
from jax.experimental import pallas as pl
from jax.experimental.pallas import tpu as pltpu
from jax.experimental.pallas import tpu_sc as plsc

_N = 131072
_K = 512
_D = 32
_BETA = 0.25
_BN = 4096
_G = _N // _BN

_NC = 2
_NS = 16
_NW = _NC * _NS
_NH = _N // 2
_BPW = _NH // _NW
_CH = 1024
_NCH = _BPW // _CH


def _dist_body(z_ref, emb_ref, inds_ref, loss_ref):
    i = pl.program_id(0)
    z = z_ref[...]
    emb = emb_ref[...]
    e_sq = jnp.sum(emb * emb, axis=1, keepdims=True)
    z_sq = jnp.sum(z * z, axis=1, keepdims=True)
    z_sq_row = jax.lax.transpose(z_sq, (1, 0))
    dist = (z_sq_row + e_sq) - 2.0 * jax.lax.dot_general(
        emb.astype(jnp.bfloat16), z.astype(jnp.bfloat16),
        (((1,), (1,)), ((), ())),
        preferred_element_type=jnp.float32)
    row = jax.lax.broadcasted_iota(jnp.int32, (_K, _BN), 0)
    dmin = jnp.min(dist, axis=0)
    inds = jnp.min(jnp.where(dist == dmin[None, :], row, _K),
                   axis=0).astype(jnp.int32)
    inds_ref[0, 0, :] = inds
    partial = jnp.sum(dmin).reshape(1, 1)

    @pl.when(i == 0)
    def _():
        loss_ref[...] = jnp.zeros((1, 1), jnp.float32)

    loss_ref[...] += partial


_NSPLIT = 2
_G2 = _G // _NSPLIT


def _make_tc_dist(split):
    return pl.pallas_call(
        _dist_body,
        grid=(_G2,),
        in_specs=[
            pl.BlockSpec((_BN, _D), lambda i: (i + split * _G2, 0)),
            pl.BlockSpec((_K, _D), lambda i: (0, 0)),
        ],
        out_specs=[
            pl.BlockSpec((1, 1, _BN), lambda i: (i, 0, 0)),
            pl.BlockSpec((1, 1), lambda i: (0, 0)),
        ],
        out_shape=[
            jax.ShapeDtypeStruct((_G2, 1, _BN), jnp.int32),
            jax.ShapeDtypeStruct((1, 1), jnp.float32),
        ],
    )


_tc_dists = [_make_tc_dist(s) for s in range(_NSPLIT)]


def _gather_body(emb_hbm, idx_hbm, out_hbm, idx_v, rows_v, sem_i, sem_g, sem_s):
    wid = lax.axis_index("s") * _NC + lax.axis_index("c")
    base = wid * _BPW
    pltpu.async_copy(idx_hbm.at[pl.ds(base, _CH)], idx_v.at[0], sem_i).wait()
    store_handles = []
    for c in range(_NCH):
        cur = c % 2
        if c + 1 < _NCH:
            nxt_off = base + (c + 1) * _CH
            h_idx = pltpu.async_copy(
                idx_hbm.at[pl.ds(nxt_off, _CH)], idx_v.at[1 - cur], sem_i)
        if c >= 2:
            store_handles[c - 2].wait()
        pltpu.async_copy(emb_hbm.at[idx_v.at[cur]], rows_v.at[cur],
                         sem_g).wait()
        store_handles.append(pltpu.async_copy(
            rows_v.at[cur], out_hbm.at[pl.ds(base + c * _CH, _CH)], sem_s))
        if c + 1 < _NCH:
            h_idx.wait()
    for h in store_handles[-2:]:
        h.wait()


_sc_gather = pl.kernel(
    _gather_body,
    out_type=jax.ShapeDtypeStruct((_NH, _D), jnp.float32),
    mesh=plsc.VectorSubcoreMesh(core_axis_name="c", subcore_axis_name="s"),
    scratch_types=[
        pltpu.VMEM((2, _CH), jnp.int32),
        pltpu.VMEM((2, _CH, _D), jnp.float32),
        pltpu.SemaphoreType.DMA,
        pltpu.SemaphoreType.DMA,
        pltpu.SemaphoreType.DMA,
    ],
    compiler_params=pltpu.CompilerParams(use_tc_tiling_on_sc=False),
)


def kernel(z_e, embeddings):
    zq_parts, inds_parts = [], []
    loss_acc = None
    for s in range(_NSPLIT):
        inds3, lacc = _tc_dists[s](z_e, embeddings)
        inds_s = inds3.reshape(_NH)
        zq_parts.append(_sc_gather(embeddings, inds_s))
        inds_parts.append(inds_s)
        loss_acc = lacc if loss_acc is None else loss_acc + lacc
    zq = jnp.concatenate(zq_parts, axis=0)
    inds = jnp.concatenate(inds_parts, axis=0)
    loss = loss_acc[0, 0] * ((1.0 + _BETA) / (_N * _D))
    return (zq, inds, loss)
